# baseline (speedup 1.0000x reference)
import functools

import jax
import jax.numpy as jnp
from jax import lax
from jax.experimental import pallas as pl
from jax.experimental.pallas import tpu as pltpu
from jax.experimental.pallas import tpu_sc as plsc

BS = 4096
SEQ = 16
D = 512
ROW = 4 * D
SPLIT = 1024

_info = plsc.get_sparse_core_info()
_NC, _NS = _info.num_cores, _info.num_subcores
_NW = _NC * _NS
_CHUNK = 16


def _sc_gather(table, idx, n):
    mesh = plsc.VectorSubcoreMesh(core_axis_name="c", subcore_axis_name="s")
    b_per_w = n // _NW
    n_chunks = b_per_w // _CHUNK

    @functools.partial(
        pl.kernel,
        mesh=mesh,
        out_type=jax.ShapeDtypeStruct((n, ROW), jnp.float32),
        scratch_types=[
            pltpu.VMEM((n_chunks, _CHUNK), jnp.int32),
            pltpu.VMEM((_CHUNK, ROW), jnp.float32),
            pltpu.VMEM((_CHUNK, ROW), jnp.float32),
            pltpu.SemaphoreType.DMA,
            pltpu.SemaphoreType.DMA,
        ],
    )
    def gather_kernel(table_hbm, idx_hbm, out_hbm, idx_v, rows0, rows1, sem0, sem1):
        wid = lax.axis_index("s") * _NC + lax.axis_index("c")
        base = wid * b_per_w
        for c in range(n_chunks):
            pltpu.sync_copy(idx_hbm.at[pl.ds(base + c * _CHUNK, _CHUNK)], idx_v.at[c])
        bufs = (rows0, rows1)
        sems = (sem0, sem1)
        copies = [None, None]
        for c in range(n_chunks):
            s = c % 2
            copies[s] = pltpu.make_async_copy(
                table_hbm.at[idx_v[c, :]], bufs[s], sems[s]
            )
            copies[s].start()
            if c >= 1:
                p = (c - 1) % 2
                copies[p].wait()
                pltpu.sync_copy(
                    bufs[p], out_hbm.at[pl.ds(base + (c - 1) * _CHUNK, _CHUNK)]
                )
        last = (n_chunks - 1) % 2
        copies[last].wait()
        pltpu.sync_copy(
            bufs[last], out_hbm.at[pl.ds(base + (n_chunks - 1) * _CHUNK, _CHUNK)]
        )

    return gather_kernel(table, idx)


_B_BLK = 256


def _tc_body(x_ref, w_ref, e_ref, o_ref):
    o_ref[...] = x_ref[...]
    wk = w_ref[...].reshape(_B_BLK, 4, D)
    scale = (e_ref[...] * (4.0 / 22.0)).reshape(_B_BLK, 1, 1)
    o_ref[:, 7:11, :] = x_ref[:, 7:11, :] + wk * scale


def _tc_body_aliased(o1_ref, x_ref, w_ref, e_ref, o_ref):
    del o1_ref
    _tc_body(x_ref, w_ref, e_ref, o_ref)


def _tc_add_part0(x, walks_h, eps2, n):
    return pl.pallas_call(
        _tc_body,
        grid=(n // _B_BLK,),
        in_specs=[
            pl.BlockSpec((_B_BLK, SEQ, D), lambda i: (i, 0, 0)),
            pl.BlockSpec((_B_BLK, ROW), lambda i: (i, 0)),
            pl.BlockSpec((_B_BLK, 1), lambda i: (i, 0)),
        ],
        out_specs=pl.BlockSpec((_B_BLK, SEQ, D), lambda i: (i, 0, 0)),
        out_shape=jax.ShapeDtypeStruct((BS, SEQ, D), jnp.float32),
    )(x, walks_h, eps2)


def _tc_add_part1(prev, x, walks_h, eps2, start, n):
    off = start // _B_BLK
    return pl.pallas_call(
        _tc_body_aliased,
        grid=(n // _B_BLK,),
        in_specs=[
            pl.BlockSpec(memory_space=pl.ANY),
            pl.BlockSpec((_B_BLK, SEQ, D), lambda i: (i + off, 0, 0)),
            pl.BlockSpec((_B_BLK, ROW), lambda i: (i, 0)),
            pl.BlockSpec((_B_BLK, 1), lambda i: (i + off, 0)),
        ],
        out_specs=pl.BlockSpec((_B_BLK, SEQ, D), lambda i: (i + off, 0, 0)),
        out_shape=jax.ShapeDtypeStruct((BS, SEQ, D), jnp.float32),
        input_output_aliases={0: 0},
    )(prev, x, walks_h, eps2)


def kernel(x, w, eps, log_mat_half):
    w = w.astype(jnp.int32)
    eps2 = eps.reshape(BS, 1)
    parts = (1024, 1024, 2048)
    starts = (0, 1024, 2048)
    walks = [
        _sc_gather(log_mat_half, lax.slice(w, (s,), (s + n,)), n)
        for s, n in zip(starts, parts)
    ]
    out = _tc_add_part0(x, walks[0], eps2, parts[0])
    for s, n, wk in zip(starts[1:], parts[1:], walks[1:]):
        out = _tc_add_part1(out, x, wk, eps2, s, n)
    return out


# speedup vs baseline: 1.0067x; 1.0067x over previous
import functools

import jax
import jax.numpy as jnp
from jax import lax
from jax.experimental import pallas as pl
from jax.experimental.pallas import tpu as pltpu
from jax.experimental.pallas import tpu_sc as plsc

BS = 4096
SEQ = 16
D = 512
ROW = 4 * D
SPLIT = 1024

_info = plsc.get_sparse_core_info()
_NC, _NS = _info.num_cores, _info.num_subcores
_NW = _NC * _NS
_CHUNK = 16


def _sc_gather(table, idx, n):
    mesh = plsc.VectorSubcoreMesh(core_axis_name="c", subcore_axis_name="s")
    b_per_w = n // _NW
    n_chunks = b_per_w // _CHUNK

    @functools.partial(
        pl.kernel,
        mesh=mesh,
        out_type=jax.ShapeDtypeStruct((n, ROW), jnp.float32),
        scratch_types=[
            pltpu.VMEM((n_chunks, _CHUNK), jnp.int32),
            pltpu.VMEM((_CHUNK, ROW), jnp.float32),
            pltpu.VMEM((_CHUNK, ROW), jnp.float32),
            pltpu.SemaphoreType.DMA,
            pltpu.SemaphoreType.DMA,
        ],
    )
    def gather_kernel(table_hbm, idx_hbm, out_hbm, idx_v, rows0, rows1, sem0, sem1):
        wid = lax.axis_index("s") * _NC + lax.axis_index("c")
        base = wid * b_per_w
        for c in range(n_chunks):
            pltpu.sync_copy(idx_hbm.at[pl.ds(base + c * _CHUNK, _CHUNK)], idx_v.at[c])
        bufs = (rows0, rows1)
        sems = (sem0, sem1)
        copies = [None, None]
        for c in range(n_chunks):
            s = c % 2
            copies[s] = pltpu.make_async_copy(
                table_hbm.at[idx_v[c, :]], bufs[s], sems[s]
            )
            copies[s].start()
            if c >= 1:
                p = (c - 1) % 2
                copies[p].wait()
                pltpu.sync_copy(
                    bufs[p], out_hbm.at[pl.ds(base + (c - 1) * _CHUNK, _CHUNK)]
                )
        last = (n_chunks - 1) % 2
        copies[last].wait()
        pltpu.sync_copy(
            bufs[last], out_hbm.at[pl.ds(base + (n_chunks - 1) * _CHUNK, _CHUNK)]
        )

    return gather_kernel(table, idx)


_B_BLK = 256


def _tc_body(x_ref, w_ref, e_ref, o_ref):
    o_ref[...] = x_ref[...]
    wk = w_ref[...].reshape(_B_BLK, 4, D)
    scale = (e_ref[...] * (4.0 / 22.0)).reshape(_B_BLK, 1, 1)
    o_ref[:, 7:11, :] = x_ref[:, 7:11, :] + wk * scale


def _tc_body_aliased(o1_ref, x_ref, w_ref, e_ref, o_ref):
    del o1_ref
    _tc_body(x_ref, w_ref, e_ref, o_ref)


def _tc_add_part0(x, walks_h, eps2, n):
    return pl.pallas_call(
        _tc_body,
        grid=(n // _B_BLK,),
        in_specs=[
            pl.BlockSpec((_B_BLK, SEQ, D), lambda i: (i, 0, 0)),
            pl.BlockSpec((_B_BLK, ROW), lambda i: (i, 0)),
            pl.BlockSpec((_B_BLK, 1), lambda i: (i, 0)),
        ],
        out_specs=pl.BlockSpec((_B_BLK, SEQ, D), lambda i: (i, 0, 0)),
        out_shape=jax.ShapeDtypeStruct((BS, SEQ, D), jnp.float32),
    )(x, walks_h, eps2)


def _tc_add_part1(prev, x, walks_h, eps2, start, n):
    off = start // _B_BLK
    return pl.pallas_call(
        _tc_body_aliased,
        grid=(n // _B_BLK,),
        in_specs=[
            pl.BlockSpec(memory_space=pl.ANY),
            pl.BlockSpec((_B_BLK, SEQ, D), lambda i: (i + off, 0, 0)),
            pl.BlockSpec((_B_BLK, ROW), lambda i: (i, 0)),
            pl.BlockSpec((_B_BLK, 1), lambda i: (i + off, 0)),
        ],
        out_specs=pl.BlockSpec((_B_BLK, SEQ, D), lambda i: (i + off, 0, 0)),
        out_shape=jax.ShapeDtypeStruct((BS, SEQ, D), jnp.float32),
        input_output_aliases={0: 0},
    )(prev, x, walks_h, eps2)


def kernel(x, w, eps, log_mat_half):
    w = w.astype(jnp.int32)
    eps2 = eps.reshape(BS, 1)
    parts = (1024, 3072)
    starts = (0, 1024)
    walks = [
        _sc_gather(log_mat_half, lax.slice(w, (s,), (s + n,)), n)
        for s, n in zip(starts, parts)
    ]
    out = _tc_add_part0(x, walks[0], eps2, parts[0])
    for s, n, wk in zip(starts[1:], parts[1:], walks[1:]):
        out = _tc_add_part1(out, x, wk, eps2, s, n)
    return out
